# baseline (speedup 1.0000x reference)
import functools

import jax
import jax.numpy as jnp
from jax import lax
from jax.experimental import pallas as pl
from jax.experimental.pallas import tpu as pltpu
from jax.experimental.pallas import tpu_sc as plsc

N_USERS = 25000
N_ITEMS = 25000
N_NODES = N_USERS + N_ITEMS
E = 800000
D = 64
NL = 3
B = 4096

NC = 2
NS = 16
L = 16

HALF = N_NODES // NC
ACC_ROWS = 25600
K = 400
KS = 80
NSUB = K // KS
CHUNKS = E // (NS * K)
WB = 200
WB_CHUNKS = HALF // WB
NFULL = WB // L

_mesh = plsc.VectorSubcoreMesh(core_axis_name="c", subcore_axis_name="s")
_params = pltpu.CompilerParams(use_tc_tiling_on_sc=False,
                               needs_layout_passes=False)


def _rsqrt16(x):
    i = plsc.bitcast(x, jnp.int32)
    y = plsc.bitcast(jnp.int32(0x5F3759DF) - (i >> 1), jnp.float32)
    for _ in range(3):
        y = y * (jnp.float32(1.5) - jnp.float32(0.5) * x * y * y)
    return jnp.where(x > 0, y, jnp.float32(0.0))


def _localize(idx_ref, p, base_node):
    for j in range(NSUB):
        for g in range(KS // L):
            sl = pl.ds(g * L, L)
            v = idx_ref[p, j, sl] - base_node
            ok = (v >= 0) & (v < HALF)
            idx_ref[p, j, sl] = jnp.where(ok, v, HALF)


def _deg_body(srcr, dstr, t0, d_out, g0_out,
              src_i, dst_i, ones, degbuf, tin, tout, dv, degacc, si, ss):
    c = lax.axis_index("c")
    s = lax.axis_index("s")
    base_node = c * HALF

    def _orow(r, _):
        ones[r, pl.ds(0, L)] = jnp.ones((L,), jnp.float32)
        return 0
    lax.fori_loop(0, KS, _orow, 0)

    def _zrow(r, _):
        degbuf[r, pl.ds(0, L)] = jnp.zeros((L,), jnp.float32)
        return 0
    lax.fori_loop(0, WB, _zrow, 0)
    zd = [pltpu.async_copy(degbuf, degacc.at[pl.ds(s * 1600 + q * WB, WB)], ss)
          for q in range(1600 // WB)]
    for dsc in zd:
        dsc.wait()
    plsc.subcore_barrier()

    def _drain10():
        for _ in range(2 * NSUB):
            pltpu.make_async_copy(ones, degacc.at[dst_i.at[0, 0]], ss).wait()

    def _chunk(ci, p, first, fire_pred):
        row = s * CHUNKS + ci
        pltpu.make_async_copy(srcr.at[row], src_i.at[p], si).wait()
        pltpu.make_async_copy(dstr.at[row], dst_i.at[p], si).wait()
        _localize(src_i, p, base_node)
        _localize(dst_i, p, base_node)
        if not first:
            _drain10()
        if fire_pred is None:
            pltpu.async_copy(srcr.at[row + 1], src_i.at[1 - p], si)
            pltpu.async_copy(dstr.at[row + 1], dst_i.at[1 - p], si)
        else:
            @pl.when(fire_pred)
            def _():
                pltpu.async_copy(srcr.at[row + 1], src_i.at[1 - p], si)
                pltpu.async_copy(dstr.at[row + 1], dst_i.at[1 - p], si)
        for j in range(NSUB):
            pltpu.async_copy(ones, degacc.at[src_i.at[p, j]], ss, add=True)
            pltpu.async_copy(ones, degacc.at[dst_i.at[p, j]], ss, add=True)

    row0 = s * CHUNKS
    pltpu.async_copy(srcr.at[row0], src_i.at[0], si)
    pltpu.async_copy(dstr.at[row0], dst_i.at[0], si)
    _chunk(0, 0, True, None)

    def _pair(i, _):
        _chunk(2 * i + 1, 1, False, None)
        _chunk(2 * i + 2, 0, False, i < (CHUNKS - 1) // 2 - 1)
        return 0

    lax.fori_loop(0, (CHUNKS - 1) // 2, _pair, 0)
    _drain10()
    plsc.subcore_barrier()

    def _wchunk(i, _):
        wc = s + NS * i

        @pl.when(wc < WB_CHUNKS)
        def _():
            gbase = base_node + wc * WB
            pltpu.sync_copy(degacc.at[pl.ds(wc * WB, WB)], degbuf)
            pltpu.sync_copy(t0.at[pl.ds(gbase, WB)], tin)

            def _group(off, lanes):
                rowv = off + lax.iota(jnp.int32, L)
                deg16 = plsc.load_gather(degbuf, [rowv, jnp.zeros((L,), jnp.int32)])
                d16 = _rsqrt16(deg16)
                dv[pl.ds(off, L)] = d16
                for t in lanes:
                    r = off + t
                    dt = d16[t]
                    for q in range(D // L):
                        sl = pl.ds(q * L, L)
                        tout[r, sl] = tin[r, sl] * dt

            def _fullg(g, _):
                _group(g * L, range(L))
                return 0
            lax.fori_loop(0, NFULL, _fullg, 0)
            _group(WB - L, range(L - (WB % L or L), L))
            pltpu.sync_copy(dv, d_out.at[pl.ds(gbase, WB)])
            pltpu.sync_copy(tout, g0_out.at[pl.ds(gbase, WB)])
        return 0

    lax.fori_loop(0, (WB_CHUNKS + NS - 1) // NS, _wchunk, 0)


_deg_call = functools.partial(
    pl.kernel,
    out_type=(jax.ShapeDtypeStruct((N_NODES,), jnp.float32),
              jax.ShapeDtypeStruct((N_NODES, D), jnp.float32)),
    mesh=_mesh,
    compiler_params=_params,
    scratch_types=[
        pltpu.VMEM((2, NSUB, KS), jnp.int32),
        pltpu.VMEM((2, NSUB, KS), jnp.int32),
        pltpu.VMEM((KS, L), jnp.float32),
        pltpu.VMEM((WB, L), jnp.float32),
        pltpu.VMEM((WB, D), jnp.float32),
        pltpu.VMEM((WB, D), jnp.float32),
        pltpu.VMEM((WB,), jnp.float32),
        pltpu.VMEM_SHARED((ACC_ROWS, L), jnp.float32),
        pltpu.SemaphoreType.DMA,
        pltpu.SemaphoreType.DMA,
    ],
)(_deg_body)


def _layer_body(gt, srcr, dstr, d_in, t_out, g_out,
                src_i, dl, gbuf, dv, acc,
                g0, g1, g2, g3, g4, s0, s1, s2, s3, s4, si):
    c = lax.axis_index("c")
    s = lax.axis_index("s")
    base_node = c * HALF
    gsem = (g0, g1, g2, g3, g4)
    ssem = (s0, s1, s2, s3, s4)

    def _slot(j):
        return gbuf.at[pl.ds(j * KS, KS)]

    def _zrow(r, _):
        for q in range(D // L):
            gbuf[r, pl.ds(q * L, L)] = jnp.zeros((L,), jnp.float32)
        return 0
    lax.fori_loop(0, K, _zrow, 0)
    zd = [pltpu.async_copy(gbuf, acc.at[pl.ds(s * 1600 + q * K, K)], s0)
          for q in range(1600 // K)]
    for dsc in zd:
        dsc.wait()
    plsc.subcore_barrier()

    def _chunk(ci, p, first, fire_pred):
        row = s * CHUNKS + ci
        pltpu.make_async_copy(srcr.at[row], src_i.at[p], si).wait()
        pltpu.make_async_copy(dstr.at[row], dl.at[p], si).wait()
        _localize(dl, p, base_node)
        for j in range(NSUB):
            if not first:
                pltpu.make_async_copy(_slot(j), acc.at[dl.at[0, 0]],
                                      ssem[j]).wait()
            pltpu.async_copy(gt.at[src_i.at[p, j]], _slot(j), gsem[j])
        if fire_pred is None:
            pltpu.async_copy(srcr.at[row + 1], src_i.at[1 - p], si)
            pltpu.async_copy(dstr.at[row + 1], dl.at[1 - p], si)
        else:
            @pl.when(fire_pred)
            def _():
                pltpu.async_copy(srcr.at[row + 1], src_i.at[1 - p], si)
                pltpu.async_copy(dstr.at[row + 1], dl.at[1 - p], si)
        for j in range(NSUB):
            pltpu.make_async_copy(gt.at[src_i.at[p, j]], _slot(j),
                                  gsem[j]).wait()
            pltpu.async_copy(_slot(j), acc.at[dl.at[p, j]], ssem[j], add=True)

    row0 = s * CHUNKS
    pltpu.async_copy(srcr.at[row0], src_i.at[0], si)
    pltpu.async_copy(dstr.at[row0], dl.at[0], si)
    _chunk(0, 0, True, None)

    def _pair(i, _):
        _chunk(2 * i + 1, 1, False, None)
        _chunk(2 * i + 2, 0, False, i < (CHUNKS - 1) // 2 - 1)
        return 0

    lax.fori_loop(0, (CHUNKS - 1) // 2, _pair, 0)
    for j in range(NSUB):
        pltpu.make_async_copy(_slot(j), acc.at[dl.at[0, 0]], ssem[j]).wait()
    plsc.subcore_barrier()

    def _wchunk(i, _):
        wc = s + NS * i

        @pl.when(wc < WB_CHUNKS)
        def _():
            gbase = base_node + wc * WB
            pltpu.sync_copy(acc.at[pl.ds(wc * WB, WB)], gbuf.at[pl.ds(0, WB)])
            pltpu.sync_copy(d_in.at[pl.ds(gbase, WB)], dv)

            def _group(off, lanes):
                d16 = dv[pl.ds(off, L)]
                for t in lanes:
                    r = off + t
                    dt = d16[t]
                    for q in range(D // L):
                        sl = pl.ds(q * L, L)
                        tval = gbuf[r, sl] * dt
                        gbuf[WB + r, sl] = tval
                        gbuf[r, sl] = tval * dt

            def _fullg(g, _):
                _group(g * L, range(L))
                return 0
            lax.fori_loop(0, NFULL, _fullg, 0)
            _group(WB - L, range(L - (WB % L or L), L))
            pltpu.sync_copy(gbuf.at[pl.ds(WB, WB)], t_out.at[pl.ds(gbase, WB)])
            pltpu.sync_copy(gbuf.at[pl.ds(0, WB)], g_out.at[pl.ds(gbase, WB)])
        return 0

    lax.fori_loop(0, (WB_CHUNKS + NS - 1) // NS, _wchunk, 0)


_layer_call = functools.partial(
    pl.kernel,
    out_type=(jax.ShapeDtypeStruct((N_NODES, D), jnp.float32),
              jax.ShapeDtypeStruct((N_NODES, D), jnp.float32)),
    mesh=_mesh,
    compiler_params=_params,
    scratch_types=[
        pltpu.VMEM((2, NSUB, KS), jnp.int32),
        pltpu.VMEM((2, NSUB, KS), jnp.int32),
        pltpu.VMEM((K, D), jnp.float32),
        pltpu.VMEM((WB,), jnp.float32),
        pltpu.VMEM_SHARED((ACC_ROWS, D), jnp.float32),
        pltpu.SemaphoreType.DMA, pltpu.SemaphoreType.DMA,
        pltpu.SemaphoreType.DMA, pltpu.SemaphoreType.DMA,
        pltpu.SemaphoreType.DMA,
        pltpu.SemaphoreType.DMA, pltpu.SemaphoreType.DMA,
        pltpu.SemaphoreType.DMA, pltpu.SemaphoreType.DMA,
        pltpu.SemaphoreType.DMA,
        pltpu.SemaphoreType.DMA,
    ],
)(_layer_body)


BPW = B // (NC * NS)


def _final_body(t0, t1, t2, t3, users, items, gamma,
                uidx, iidx, rbuf, usum, isum, gout, sem):
    c = lax.axis_index("c")
    s = lax.axis_index("s")
    wid = s * NC + c
    base = wid * BPW
    pltpu.sync_copy(users.at[pl.ds(base, BPW)], uidx)
    pltpu.sync_copy(items.at[pl.ds(base, BPW)], iidx)
    for g in range(BPW // L):
        sl = pl.ds(g * L, L)
        iidx[sl] = iidx[sl] + N_USERS

    def _acc_rows(idx, dst):
        pltpu.async_copy(t0.at[idx], dst, sem).wait()
        for t in (t1, t2, t3):
            pltpu.async_copy(t.at[idx], rbuf, sem).wait()

            def _add(r, _):
                for j in range(D // L):
                    sl = pl.ds(j * L, L)
                    dst[r, sl] = dst[r, sl] + rbuf[r, sl]
                return 0
            lax.fori_loop(0, BPW, _add, 0)

    _acc_rows(uidx, usum)
    _acc_rows(iidx, isum)

    def _dot(g, _):
        rowv = g * L + lax.iota(jnp.int32, L)
        acc16 = jnp.zeros((L,), jnp.float32)

        def _dim(dd, a):
            colv = jnp.full((L,), dd, jnp.int32)
            cu = plsc.load_gather(usum, [rowv, colv])
            ci = plsc.load_gather(isum, [rowv, colv])
            return a + cu * ci
        acc16 = lax.fori_loop(0, D, _dim, acc16)
        gout[pl.ds(g * L, L)] = acc16 * jnp.float32(1.0 / ((NL + 1) * (NL + 1)))
        return 0
    lax.fori_loop(0, BPW // L, _dot, 0)
    pltpu.sync_copy(gout, gamma.at[pl.ds(base, BPW)])


_final_call = functools.partial(
    pl.kernel,
    out_type=jax.ShapeDtypeStruct((B,), jnp.float32),
    mesh=_mesh,
    compiler_params=_params,
    scratch_types=[
        pltpu.VMEM((BPW,), jnp.int32),
        pltpu.VMEM((BPW,), jnp.int32),
        pltpu.VMEM((BPW, D), jnp.float32),
        pltpu.VMEM((BPW, D), jnp.float32),
        pltpu.VMEM((BPW, D), jnp.float32),
        pltpu.VMEM((BPW,), jnp.float32),
        pltpu.SemaphoreType.DMA,
    ],
)(_final_body)


def kernel(users, items, edge_index, edge_vals, user_emb, item_emb):
    del edge_vals
    src = edge_index[0].astype(jnp.int32).reshape(NS * CHUNKS, NSUB, KS)
    dst = edge_index[1].astype(jnp.int32).reshape(NS * CHUNKS, NSUB, KS)
    t0 = jnp.concatenate([user_emb, item_emb], axis=0)
    d, gt = _deg_call(src, dst, t0)
    t1, g1 = _layer_call(gt, src, dst, d)
    t2, g2 = _layer_call(g1, src, dst, d)
    t3, _ = _layer_call(g2, src, dst, d)
    return _final_call(t0, t1, t2, t3,
                       users.astype(jnp.int32), items.astype(jnp.int32))


# speedup vs baseline: 2.2524x; 2.2524x over previous
import functools

import jax
import jax.numpy as jnp
from jax import lax
from jax.experimental import pallas as pl
from jax.experimental.pallas import tpu as pltpu
from jax.experimental.pallas import tpu_sc as plsc

N_USERS = 25000
N_ITEMS = 25000
N_NODES = N_USERS + N_ITEMS
E = 800000
D = 64
NL = 3
B = 4096

NC = 2
NS = 16
L = 16

DH = D // NC
K = 400
KS = 80
NSUB = K // KS
CHUNKS = E // (NS * K)
TROWS = N_NODES // NS
ZR = 125

_mesh = plsc.VectorSubcoreMesh(core_axis_name="c", subcore_axis_name="s")
_params = pltpu.CompilerParams(use_tc_tiling_on_sc=False,
                               needs_layout_passes=False)


def _layer_body(table, srcr, dstr, wr, table_out,
                src_i, dl, w_v, gbuf, sbuf, zbuf, acc, g0, g1, s0, s1, si):
    c = lax.axis_index("c")
    s = lax.axis_index("s")
    cbase = c * N_NODES
    gsem = (g0, g1)
    ssem = (s0, s1)

    def _zrow(r, _):
        for q in range(DH // L):
            zbuf[r, pl.ds(q * L, L)] = jnp.zeros((L,), jnp.float32)
        return 0
    lax.fori_loop(0, ZR, _zrow, 0)
    zd = [pltpu.async_copy(zbuf, acc.at[pl.ds(s * TROWS + q * ZR, ZR)], s0)
          for q in range(TROWS // ZR)]
    for d in zd:
        d.wait()
    plsc.subcore_barrier()

    def _drain_scatter(b):
        pltpu.make_async_copy(sbuf.at[b], acc.at[dl.at[0, 0]], ssem[b]).wait()

    def _do_chunk(ci, p, first, fire_pred):
        row = s * CHUNKS + ci
        pltpu.make_async_copy(srcr.at[row], src_i.at[p], si).wait()
        pltpu.make_async_copy(dstr.at[row], dl.at[p], si).wait()
        pltpu.make_async_copy(wr.at[row], w_v.at[p], si).wait()
        for j in range(NSUB):
            for g in range(KS // L):
                sl = pl.ds(g * L, L)
                src_i[p, j, sl] = src_i[p, j, sl] + cbase
        gat = [None] * NSUB
        for j in range(2):
            if not first:
                _drain_scatter(j)
            gat[j] = pltpu.async_copy(table.at[src_i.at[p, j]],
                                      gbuf.at[j], gsem[j])
        if fire_pred is None:
            pltpu.async_copy(srcr.at[row + 1], src_i.at[1 - p], si)
            pltpu.async_copy(dstr.at[row + 1], dl.at[1 - p], si)
            pltpu.async_copy(wr.at[row + 1], w_v.at[1 - p], si)
        else:
            @pl.when(fire_pred)
            def _():
                pltpu.async_copy(srcr.at[row + 1], src_i.at[1 - p], si)
                pltpu.async_copy(dstr.at[row + 1], dl.at[1 - p], si)
                pltpu.async_copy(wr.at[row + 1], w_v.at[1 - p], si)
        last = [None, None]
        for j in range(NSUB):
            b = j % 2
            gat[j].wait()
            if j >= 2:
                last[b].wait()

            def _mul(g, _):
                w16 = w_v[p, pl.ds(j * KS + g * L, L)]
                for t in range(L):
                    e = g * L + t
                    we = w16[t]
                    for q in range(DH // L):
                        sl = pl.ds(q * L, L)
                        sbuf[b, e, sl] = gbuf[b, e, sl] * we
                return 0
            lax.fori_loop(0, KS // L, _mul, 0)
            last[b] = pltpu.async_copy(sbuf.at[b], acc.at[dl.at[p, j]],
                                       ssem[b], add=True)
            if j + 2 < NSUB:
                gat[j + 2] = pltpu.async_copy(table.at[src_i.at[p, j + 2]],
                                              gbuf.at[b], gsem[b])
        return last

    row0 = s * CHUNKS
    pltpu.async_copy(srcr.at[row0], src_i.at[0], si)
    pltpu.async_copy(dstr.at[row0], dl.at[0], si)
    pltpu.async_copy(wr.at[row0], w_v.at[0], si)
    _do_chunk(0, 0, True, None)

    def _pair(i, _):
        _do_chunk(2 * i + 1, 1, False, None)
        _do_chunk(2 * i + 2, 0, False, i < (CHUNKS - 1) // 2 - 1)
        return 0

    lax.fori_loop(0, (CHUNKS - 1) // 2, _pair, 0)
    _drain_scatter(1)
    _drain_scatter(0)
    plsc.subcore_barrier()

    pltpu.sync_copy(acc.at[pl.ds(s * TROWS, TROWS)],
                    table_out.at[pl.ds(cbase + s * TROWS, TROWS)])


_layer_call = functools.partial(
    pl.kernel,
    out_type=jax.ShapeDtypeStruct((NC * N_NODES, DH), jnp.float32),
    mesh=_mesh,
    compiler_params=_params,
    scratch_types=[
        pltpu.VMEM((2, NSUB, KS), jnp.int32),
        pltpu.VMEM((2, NSUB, KS), jnp.int32),
        pltpu.VMEM((2, K), jnp.float32),
        pltpu.VMEM((2, KS, DH), jnp.float32),
        pltpu.VMEM((2, KS, DH), jnp.float32),
        pltpu.VMEM((ZR, DH), jnp.float32),
        pltpu.VMEM_SHARED((N_NODES, DH), jnp.float32),
        pltpu.SemaphoreType.DMA,
        pltpu.SemaphoreType.DMA,
        pltpu.SemaphoreType.DMA,
        pltpu.SemaphoreType.DMA,
        pltpu.SemaphoreType.DMA,
    ],
)(_layer_body)


BPW = B // (NC * NS)


def _final_body(t0, t1, t2, t3, users, items, gamma,
                uidx, iidx, rbuf, us0, us1, is0, is1, gout, sem):
    c = lax.axis_index("c")
    s = lax.axis_index("s")
    wid = s * NC + c
    base = wid * BPW
    pltpu.sync_copy(users.at[pl.ds(base, BPW)], uidx)
    pltpu.sync_copy(items.at[pl.ds(base, BPW)], iidx)
    for g in range(BPW // L):
        sl = pl.ds(g * L, L)
        iidx[sl] = iidx[sl] + N_USERS

    def _gather_sum(idx_ref, dst):
        pltpu.async_copy(t0.at[idx_ref], dst, sem).wait()
        for t in (t1, t2, t3):
            pltpu.async_copy(t.at[idx_ref], rbuf, sem).wait()

            def _add(r, _):
                for j in range(DH // L):
                    sl = pl.ds(j * L, L)
                    dst[r, sl] = dst[r, sl] + rbuf[r, sl]
                return 0
            lax.fori_loop(0, BPW, _add, 0)

    def _acc_side(idx_ref, d0, d1):
        _gather_sum(idx_ref, d0)
        for g in range(BPW // L):
            sl = pl.ds(g * L, L)
            idx_ref[sl] = idx_ref[sl] + N_NODES
        _gather_sum(idx_ref, d1)

    _acc_side(uidx, us0, us1)
    _acc_side(iidx, is0, is1)

    def _dot(g, _):
        rowv = g * L + lax.iota(jnp.int32, L)
        acc16 = jnp.zeros((L,), jnp.float32)

        def _dim0(dd, a):
            colv = jnp.full((L,), dd, jnp.int32)
            return a + (plsc.load_gather(us0, [rowv, colv]) *
                        plsc.load_gather(is0, [rowv, colv]))
        acc16 = lax.fori_loop(0, DH, _dim0, acc16)

        def _dim1(dd, a):
            colv = jnp.full((L,), dd, jnp.int32)
            return a + (plsc.load_gather(us1, [rowv, colv]) *
                        plsc.load_gather(is1, [rowv, colv]))
        acc16 = lax.fori_loop(0, DH, _dim1, acc16)
        gout[pl.ds(g * L, L)] = acc16 * jnp.float32(1.0 / ((NL + 1) * (NL + 1)))
        return 0
    lax.fori_loop(0, BPW // L, _dot, 0)
    pltpu.sync_copy(gout, gamma.at[pl.ds(base, BPW)])


_final_call = functools.partial(
    pl.kernel,
    out_type=jax.ShapeDtypeStruct((B,), jnp.float32),
    mesh=_mesh,
    compiler_params=_params,
    scratch_types=[
        pltpu.VMEM((BPW,), jnp.int32),
        pltpu.VMEM((BPW,), jnp.int32),
        pltpu.VMEM((BPW, DH), jnp.float32),
        pltpu.VMEM((BPW, DH), jnp.float32),
        pltpu.VMEM((BPW, DH), jnp.float32),
        pltpu.VMEM((BPW, DH), jnp.float32),
        pltpu.VMEM((BPW, DH), jnp.float32),
        pltpu.VMEM((BPW,), jnp.float32),
        pltpu.SemaphoreType.DMA,
    ],
)(_final_body)


def kernel(users, items, edge_index, edge_vals, user_emb, item_emb):
    src = edge_index[0].astype(jnp.int32).reshape(NS * CHUNKS, NSUB, KS)
    dst = edge_index[1].astype(jnp.int32).reshape(NS * CHUNKS, NSUB, KS)
    w = edge_vals.astype(jnp.float32).reshape(NS * CHUNKS, K)
    t0 = jnp.concatenate([user_emb, item_emb], axis=0)
    t0s = jnp.concatenate([t0[:, :DH], t0[:, DH:]], axis=0)
    t1 = _layer_call(t0s, src, dst, w)
    t2 = _layer_call(t1, src, dst, w)
    t3 = _layer_call(t2, src, dst, w)
    return _final_call(t0s, t1, t2, t3,
                       users.astype(jnp.int32), items.astype(jnp.int32))


# speedup vs baseline: 2.2750x; 1.0101x over previous
import functools

import jax
import jax.numpy as jnp
from jax import lax
from jax.experimental import pallas as pl
from jax.experimental.pallas import tpu as pltpu
from jax.experimental.pallas import tpu_sc as plsc

N_USERS = 25000
N_ITEMS = 25000
N_NODES = N_USERS + N_ITEMS
E = 800000
D = 64
NL = 3
B = 4096

NC = 2
NS = 16
L = 16

DH = D // NC
K = 400
KS = 80
NSUB = K // KS
CHUNKS = E // (NS * K)
TROWS = N_NODES // NS
ZR = 125

_mesh = plsc.VectorSubcoreMesh(core_axis_name="c", subcore_axis_name="s")
_params = pltpu.CompilerParams(use_tc_tiling_on_sc=False,
                               needs_layout_passes=False)


def _prop_body(t0s, srcr, dstr, wr, o1, o2, o3,
               src_i, dl, w_v, gbuf, sbuf, zbuf, acc, g0, g1, s0, s1, si):
    c = lax.axis_index("c")
    s = lax.axis_index("s")
    cbase = c * N_NODES
    gsem = (g0, g1)
    ssem = (s0, s1)

    def _zrow(r, _):
        for q in range(DH // L):
            zbuf[r, pl.ds(q * L, L)] = jnp.zeros((L,), jnp.float32)
        return 0
    lax.fori_loop(0, ZR, _zrow, 0)

    def _zero_slab():
        zd = [pltpu.async_copy(zbuf, acc.at[pl.ds(s * TROWS + q * ZR, ZR)], s0)
              for q in range(TROWS // ZR)]
        for d in zd:
            d.wait()

    def _drain_scatter(b):
        pltpu.make_async_copy(sbuf.at[b], acc.at[dl.at[0, 0]], ssem[b]).wait()

    def _do_chunk(table, ci, p, first, fire_pred):
        row = s * CHUNKS + ci
        pltpu.make_async_copy(srcr.at[row], src_i.at[p], si).wait()
        pltpu.make_async_copy(dstr.at[row], dl.at[p], si).wait()
        pltpu.make_async_copy(wr.at[row], w_v.at[p], si).wait()
        for j in range(NSUB):
            for g in range(KS // L):
                sl = pl.ds(g * L, L)
                src_i[p, j, sl] = src_i[p, j, sl] + cbase
        gat = [None] * NSUB
        for j in range(2):
            if not first:
                _drain_scatter(j)
            gat[j] = pltpu.async_copy(table.at[src_i.at[p, j]],
                                      gbuf.at[j], gsem[j])
        if fire_pred is None:
            pltpu.async_copy(srcr.at[row + 1], src_i.at[1 - p], si)
            pltpu.async_copy(dstr.at[row + 1], dl.at[1 - p], si)
            pltpu.async_copy(wr.at[row + 1], w_v.at[1 - p], si)
        else:
            @pl.when(fire_pred)
            def _():
                pltpu.async_copy(srcr.at[row + 1], src_i.at[1 - p], si)
                pltpu.async_copy(dstr.at[row + 1], dl.at[1 - p], si)
                pltpu.async_copy(wr.at[row + 1], w_v.at[1 - p], si)
        last = [None, None]
        for j in range(NSUB):
            b = j % 2
            gat[j].wait()
            if j >= 2:
                last[b].wait()

            def _mul(g, _):
                w16 = w_v[p, pl.ds(j * KS + g * L, L)]
                for t in range(L):
                    e = g * L + t
                    we = w16[t]
                    for q in range(DH // L):
                        sl = pl.ds(q * L, L)
                        sbuf[b, e, sl] = gbuf[b, e, sl] * we
                return 0
            lax.fori_loop(0, KS // L, _mul, 0)
            last[b] = pltpu.async_copy(sbuf.at[b], acc.at[dl.at[p, j]],
                                       ssem[b], add=True)
            if j + 2 < NSUB:
                gat[j + 2] = pltpu.async_copy(table.at[src_i.at[p, j + 2]],
                                              gbuf.at[b], gsem[b])
        return last

    def _edge_pass(table):
        row0 = s * CHUNKS
        pltpu.async_copy(srcr.at[row0], src_i.at[0], si)
        pltpu.async_copy(dstr.at[row0], dl.at[0], si)
        pltpu.async_copy(wr.at[row0], w_v.at[0], si)
        _do_chunk(table, 0, 0, True, None)

        def _pair(i, _):
            _do_chunk(table, 2 * i + 1, 1, False, None)
            _do_chunk(table, 2 * i + 2, 0, False, i < (CHUNKS - 1) // 2 - 1)
            return 0

        lax.fori_loop(0, (CHUNKS - 1) // 2, _pair, 0)
        _drain_scatter(1)
        _drain_scatter(0)

    _zero_slab()
    plsc.subcore_barrier()
    for table, table_out, last_layer in ((t0s, o1, False), (o1, o2, False),
                                         (o2, o3, True)):
        _edge_pass(table)
        plsc.subcore_barrier()
        pltpu.sync_copy(acc.at[pl.ds(s * TROWS, TROWS)],
                        table_out.at[pl.ds(cbase + s * TROWS, TROWS)])
        if not last_layer:
            _zero_slab()
            plsc.subcore_barrier()


_prop_call = functools.partial(
    pl.kernel,
    out_type=(jax.ShapeDtypeStruct((NC * N_NODES, DH), jnp.float32),
              jax.ShapeDtypeStruct((NC * N_NODES, DH), jnp.float32),
              jax.ShapeDtypeStruct((NC * N_NODES, DH), jnp.float32)),
    mesh=_mesh,
    compiler_params=_params,
    scratch_types=[
        pltpu.VMEM((2, NSUB, KS), jnp.int32),
        pltpu.VMEM((2, NSUB, KS), jnp.int32),
        pltpu.VMEM((2, K), jnp.float32),
        pltpu.VMEM((2, KS, DH), jnp.float32),
        pltpu.VMEM((2, KS, DH), jnp.float32),
        pltpu.VMEM((ZR, DH), jnp.float32),
        pltpu.VMEM_SHARED((N_NODES, DH), jnp.float32),
        pltpu.SemaphoreType.DMA,
        pltpu.SemaphoreType.DMA,
        pltpu.SemaphoreType.DMA,
        pltpu.SemaphoreType.DMA,
        pltpu.SemaphoreType.DMA,
    ],
)(_prop_body)


BPW = B // (NC * NS)


def _final_body(t0, t1, t2, t3, users, items, gamma,
                uidx, iidx, rbuf, us0, us1, is0, is1, gout, sem):
    c = lax.axis_index("c")
    s = lax.axis_index("s")
    wid = s * NC + c
    base = wid * BPW
    pltpu.sync_copy(users.at[pl.ds(base, BPW)], uidx)
    pltpu.sync_copy(items.at[pl.ds(base, BPW)], iidx)
    for g in range(BPW // L):
        sl = pl.ds(g * L, L)
        iidx[sl] = iidx[sl] + N_USERS

    def _gather_sum(idx_ref, dst):
        pltpu.async_copy(t0.at[idx_ref], dst, sem).wait()
        for t in (t1, t2, t3):
            pltpu.async_copy(t.at[idx_ref], rbuf, sem).wait()

            def _add(r, _):
                for j in range(DH // L):
                    sl = pl.ds(j * L, L)
                    dst[r, sl] = dst[r, sl] + rbuf[r, sl]
                return 0
            lax.fori_loop(0, BPW, _add, 0)

    def _acc_side(idx_ref, d0, d1):
        _gather_sum(idx_ref, d0)
        for g in range(BPW // L):
            sl = pl.ds(g * L, L)
            idx_ref[sl] = idx_ref[sl] + N_NODES
        _gather_sum(idx_ref, d1)

    _acc_side(uidx, us0, us1)
    _acc_side(iidx, is0, is1)

    def _dot(g, _):
        rowv = g * L + lax.iota(jnp.int32, L)
        acc16 = jnp.zeros((L,), jnp.float32)

        def _dim0(dd, a):
            colv = jnp.full((L,), dd, jnp.int32)
            return a + (plsc.load_gather(us0, [rowv, colv]) *
                        plsc.load_gather(is0, [rowv, colv]))
        acc16 = lax.fori_loop(0, DH, _dim0, acc16)

        def _dim1(dd, a):
            colv = jnp.full((L,), dd, jnp.int32)
            return a + (plsc.load_gather(us1, [rowv, colv]) *
                        plsc.load_gather(is1, [rowv, colv]))
        acc16 = lax.fori_loop(0, DH, _dim1, acc16)
        gout[pl.ds(g * L, L)] = acc16 * jnp.float32(1.0 / ((NL + 1) * (NL + 1)))
        return 0
    lax.fori_loop(0, BPW // L, _dot, 0)
    pltpu.sync_copy(gout, gamma.at[pl.ds(base, BPW)])


_final_call = functools.partial(
    pl.kernel,
    out_type=jax.ShapeDtypeStruct((B,), jnp.float32),
    mesh=_mesh,
    compiler_params=_params,
    scratch_types=[
        pltpu.VMEM((BPW,), jnp.int32),
        pltpu.VMEM((BPW,), jnp.int32),
        pltpu.VMEM((BPW, DH), jnp.float32),
        pltpu.VMEM((BPW, DH), jnp.float32),
        pltpu.VMEM((BPW, DH), jnp.float32),
        pltpu.VMEM((BPW, DH), jnp.float32),
        pltpu.VMEM((BPW, DH), jnp.float32),
        pltpu.VMEM((BPW,), jnp.float32),
        pltpu.SemaphoreType.DMA,
    ],
)(_final_body)


def kernel(users, items, edge_index, edge_vals, user_emb, item_emb):
    src = edge_index[0].astype(jnp.int32).reshape(NS * CHUNKS, NSUB, KS)
    dst = edge_index[1].astype(jnp.int32).reshape(NS * CHUNKS, NSUB, KS)
    w = edge_vals.astype(jnp.float32).reshape(NS * CHUNKS, K)
    t0 = jnp.concatenate([user_emb, item_emb], axis=0)
    t0s = jnp.concatenate([t0[:, :DH], t0[:, DH:]], axis=0)
    t1, t2, t3 = _prop_call(t0s, src, dst, w)
    return _final_call(t0s, t1, t2, t3,
                       users.astype(jnp.int32), items.astype(jnp.int32))


# speedup vs baseline: 3.2044x; 1.4085x over previous
import functools

import jax
import jax.numpy as jnp
from jax import lax
from jax.experimental import pallas as pl
from jax.experimental.pallas import tpu as pltpu
from jax.experimental.pallas import tpu_sc as plsc

N_USERS = 25000
N_ITEMS = 25000
N_NODES = N_USERS + N_ITEMS
E = 800000
D = 64
NL = 3
B = 4096

NC = 2
NS = 16
L = 16

DH = D // NC
K = 400
KS = 80
NSUB = K // KS
CHUNKS = E // (NS * K)
TROWS = N_NODES // NS
ZR = 125

_mesh = plsc.VectorSubcoreMesh(core_axis_name="c", subcore_axis_name="s")
_params = pltpu.CompilerParams(use_tc_tiling_on_sc=False,
                               needs_layout_passes=False)


def _prop_body(t0s, srcr, dstr, wr, o1, o2, o3,
               src_i, dl, w_v, gbuf, zbuf, acc,
               g0, g1, g2, g3, g4, s0, s1, s2, s3, s4, si):
    c = lax.axis_index("c")
    s = lax.axis_index("s")
    cbase = c * N_NODES
    gsem = (g0, g1, g2, g3, g4)
    ssem = (s0, s1, s2, s3, s4)

    def _zrow(r, _):
        for q in range(DH // L):
            zbuf[r, pl.ds(q * L, L)] = jnp.zeros((L,), jnp.float32)
        return 0
    lax.fori_loop(0, ZR, _zrow, 0)

    def _zero_slab():
        zd = [pltpu.async_copy(zbuf, acc.at[pl.ds(s * TROWS + q * ZR, ZR)], s0)
              for q in range(TROWS // ZR)]
        for d in zd:
            d.wait()

    def _slot(j):
        return gbuf.at[j]

    def _drain_scatter(j):
        pltpu.make_async_copy(_slot(j), acc.at[dl.at[0, 0]], ssem[j]).wait()

    def _do_chunk(table, ci, p, first, fire_pred):
        row = s * CHUNKS + ci
        pltpu.make_async_copy(srcr.at[row], src_i.at[p], si).wait()
        pltpu.make_async_copy(dstr.at[row], dl.at[p], si).wait()
        pltpu.make_async_copy(wr.at[row], w_v.at[p], si).wait()
        for j in range(NSUB):
            for g in range(KS // L):
                sl = pl.ds(g * L, L)
                src_i[p, j, sl] = src_i[p, j, sl] + cbase
        if not first:
            for j in range(NSUB):
                _drain_scatter(j)
        gat = [pltpu.async_copy(table.at[src_i.at[p, j]], _slot(j), gsem[j])
               for j in range(NSUB)]
        if fire_pred is None:
            pltpu.async_copy(srcr.at[row + 1], src_i.at[1 - p], si)
            pltpu.async_copy(dstr.at[row + 1], dl.at[1 - p], si)
            pltpu.async_copy(wr.at[row + 1], w_v.at[1 - p], si)
        else:
            @pl.when(fire_pred)
            def _():
                pltpu.async_copy(srcr.at[row + 1], src_i.at[1 - p], si)
                pltpu.async_copy(dstr.at[row + 1], dl.at[1 - p], si)
                pltpu.async_copy(wr.at[row + 1], w_v.at[1 - p], si)
        for j in range(NSUB):
            gat[j].wait()

            def _mul(g, _):
                w16 = w_v[p, pl.ds(j * KS + g * L, L)]
                for t in range(L):
                    e = g * L + t
                    we = w16[t]
                    for q in range(DH // L):
                        sl = pl.ds(q * L, L)
                        gbuf[j, e, sl] = gbuf[j, e, sl] * we
                return 0
            lax.fori_loop(0, KS // L, _mul, 0)
            pltpu.async_copy(_slot(j), acc.at[dl.at[p, j]], ssem[j], add=True)

    def _edge_pass(table):
        row0 = s * CHUNKS
        pltpu.async_copy(srcr.at[row0], src_i.at[0], si)
        pltpu.async_copy(dstr.at[row0], dl.at[0], si)
        pltpu.async_copy(wr.at[row0], w_v.at[0], si)
        _do_chunk(table, 0, 0, True, None)

        def _pair(i, _):
            _do_chunk(table, 2 * i + 1, 1, False, None)
            _do_chunk(table, 2 * i + 2, 0, False, i < (CHUNKS - 1) // 2 - 1)
            return 0

        lax.fori_loop(0, (CHUNKS - 1) // 2, _pair, 0)
        for j in range(NSUB):
            _drain_scatter(j)

    _zero_slab()
    plsc.subcore_barrier()
    for table, table_out, last_layer in ((t0s, o1, False), (o1, o2, False),
                                         (o2, o3, True)):
        _edge_pass(table)
        plsc.subcore_barrier()
        pltpu.sync_copy(acc.at[pl.ds(s * TROWS, TROWS)],
                        table_out.at[pl.ds(cbase + s * TROWS, TROWS)])
        if not last_layer:
            _zero_slab()
            plsc.subcore_barrier()


_prop_call = functools.partial(
    pl.kernel,
    out_type=(jax.ShapeDtypeStruct((NC * N_NODES, DH), jnp.float32),
              jax.ShapeDtypeStruct((NC * N_NODES, DH), jnp.float32),
              jax.ShapeDtypeStruct((NC * N_NODES, DH), jnp.float32)),
    mesh=_mesh,
    compiler_params=_params,
    scratch_types=[
        pltpu.VMEM((2, NSUB, KS), jnp.int32),
        pltpu.VMEM((2, NSUB, KS), jnp.int32),
        pltpu.VMEM((2, K), jnp.float32),
        pltpu.VMEM((NSUB, KS, DH), jnp.float32),
        pltpu.VMEM((ZR, DH), jnp.float32),
        pltpu.VMEM_SHARED((N_NODES, DH), jnp.float32),
        pltpu.SemaphoreType.DMA, pltpu.SemaphoreType.DMA,
        pltpu.SemaphoreType.DMA, pltpu.SemaphoreType.DMA,
        pltpu.SemaphoreType.DMA,
        pltpu.SemaphoreType.DMA, pltpu.SemaphoreType.DMA,
        pltpu.SemaphoreType.DMA, pltpu.SemaphoreType.DMA,
        pltpu.SemaphoreType.DMA,
        pltpu.SemaphoreType.DMA,
    ],
)(_prop_body)


BPW = B // (NC * NS)


def _final_body(t0, t1, t2, t3, users, items, gamma,
                uidx, iidx, rbuf, us0, us1, is0, is1, gout, sem):
    c = lax.axis_index("c")
    s = lax.axis_index("s")
    wid = s * NC + c
    base = wid * BPW
    pltpu.sync_copy(users.at[pl.ds(base, BPW)], uidx)
    pltpu.sync_copy(items.at[pl.ds(base, BPW)], iidx)
    for g in range(BPW // L):
        sl = pl.ds(g * L, L)
        iidx[sl] = iidx[sl] + N_USERS

    def _gather_sum(idx_ref, dst):
        pltpu.async_copy(t0.at[idx_ref], dst, sem).wait()
        for t in (t1, t2, t3):
            pltpu.async_copy(t.at[idx_ref], rbuf, sem).wait()

            def _add(r, _):
                for j in range(DH // L):
                    sl = pl.ds(j * L, L)
                    dst[r, sl] = dst[r, sl] + rbuf[r, sl]
                return 0
            lax.fori_loop(0, BPW, _add, 0)

    def _acc_side(idx_ref, d0, d1):
        _gather_sum(idx_ref, d0)
        for g in range(BPW // L):
            sl = pl.ds(g * L, L)
            idx_ref[sl] = idx_ref[sl] + N_NODES
        _gather_sum(idx_ref, d1)

    _acc_side(uidx, us0, us1)
    _acc_side(iidx, is0, is1)

    def _dot(g, _):
        rowv = g * L + lax.iota(jnp.int32, L)
        acc16 = jnp.zeros((L,), jnp.float32)

        def _dim0(dd, a):
            colv = jnp.full((L,), dd, jnp.int32)
            return a + (plsc.load_gather(us0, [rowv, colv]) *
                        plsc.load_gather(is0, [rowv, colv]))
        acc16 = lax.fori_loop(0, DH, _dim0, acc16)

        def _dim1(dd, a):
            colv = jnp.full((L,), dd, jnp.int32)
            return a + (plsc.load_gather(us1, [rowv, colv]) *
                        plsc.load_gather(is1, [rowv, colv]))
        acc16 = lax.fori_loop(0, DH, _dim1, acc16)
        gout[pl.ds(g * L, L)] = acc16 * jnp.float32(1.0 / ((NL + 1) * (NL + 1)))
        return 0
    lax.fori_loop(0, BPW // L, _dot, 0)
    pltpu.sync_copy(gout, gamma.at[pl.ds(base, BPW)])


_final_call = functools.partial(
    pl.kernel,
    out_type=jax.ShapeDtypeStruct((B,), jnp.float32),
    mesh=_mesh,
    compiler_params=_params,
    scratch_types=[
        pltpu.VMEM((BPW,), jnp.int32),
        pltpu.VMEM((BPW,), jnp.int32),
        pltpu.VMEM((BPW, DH), jnp.float32),
        pltpu.VMEM((BPW, DH), jnp.float32),
        pltpu.VMEM((BPW, DH), jnp.float32),
        pltpu.VMEM((BPW, DH), jnp.float32),
        pltpu.VMEM((BPW, DH), jnp.float32),
        pltpu.VMEM((BPW,), jnp.float32),
        pltpu.SemaphoreType.DMA,
    ],
)(_final_body)


def kernel(users, items, edge_index, edge_vals, user_emb, item_emb):
    src = edge_index[0].astype(jnp.int32).reshape(NS * CHUNKS, NSUB, KS)
    dst = edge_index[1].astype(jnp.int32).reshape(NS * CHUNKS, NSUB, KS)
    w = edge_vals.astype(jnp.float32).reshape(NS * CHUNKS, K)
    t0 = jnp.concatenate([user_emb, item_emb], axis=0)
    t0s = jnp.concatenate([t0[:, :DH], t0[:, DH:]], axis=0)
    t1, t2, t3 = _prop_call(t0s, src, dst, w)
    return _final_call(t0s, t1, t2, t3,
                       users.astype(jnp.int32), items.astype(jnp.int32))
